# Initial kernel scaffold; baseline (speedup 1.0000x reference)
import jax
import jax.numpy as jnp
from jax.experimental import pallas as pl


def kernel(x, edge_index_fine, edge_src_down, edge_dst_down, edge_index_coarse, batch_size, W_d1, W_d2, W_dd, W_e1, W_e2, W_ed, W_e3, W_e4, W_sub, W_proxy, b_proxy, W_dec):
    raise NotImplementedError("write your pallas kernel here")


# speedup vs baseline: 7.7965x; 7.7965x over previous
import functools

import jax
import jax.numpy as jnp
from jax import lax
from jax.experimental import pallas as pl
from jax.experimental.pallas import tpu as pltpu
from jax.experimental.pallas import tpu_sc as plsc

_F32 = jnp.float32


def _ceil_to(a, b):
  return -(-a // b) * b


def _sc_pass(tab_l, tab_r, src2d, dst2d, *, width, n_out, split_edges=False):
  e_total = src2d.shape[0] * 128
  n_workers = 32 if split_edges else 16
  epw = e_total // n_workers
  n_iter = epw // 1024
  zr = 512
  rows_per_sub = -(-(n_out + 1) // 16)
  nz = -(-rows_per_sub // zr)
  wpr = nz * zr
  op = 16 * wpr

  if width == 1:
    rows_t = pltpu.VMEM((128,), _F32)
    zbuf_t = pltpu.VMEM((zr,), _F32)
    spm_t = pltpu.VMEM_SHARED((op,), _F32)
    out_sds = jax.ShapeDtypeStruct((op,), _F32)
  else:
    rows_t = pltpu.VMEM((128, width), _F32)
    zbuf_t = pltpu.VMEM((zr, width), _F32)
    spm_t = pltpu.VMEM_SHARED((op, width), _F32)
    out_sds = jax.ShapeDtypeStruct((op, width), _F32)

  def body(tab_l_r, tab_r_r, src_r, dst_r, out_l_r, out_r_r,
           srcv, dstv, rows, zbuf, spm, sem):
    c = lax.axis_index("c")
    s = lax.axis_index("s")

    if width == 1:
      def _zb(i, carry):
        zbuf[pl.ds(16 * i, 16)] = jnp.zeros((16,), _F32)
        return carry
      lax.fori_loop(0, zr // 16, _zb, 0)
    else:
      def _zb(i, carry):
        zbuf[i, :] = jnp.zeros((16,), _F32)
        return carry
      lax.fori_loop(0, zr, _zb, 0)

    def run(tab_ref, out_ref):
      def _zc(z, carry):
        pltpu.sync_copy(zbuf, spm.at[pl.ds((s * nz + z) * zr, zr)])
        return carry
      lax.fori_loop(0, nz, _zc, 0)
      plsc.subcore_barrier()

      wid = c * 16 + s if split_edges else s
      base_row = wid * (epw // 128)

      def _it(i, carry):
        row0 = base_row + i * 8
        pltpu.sync_copy(src_r.at[pl.ds(row0, 8)], srcv)
        pltpu.sync_copy(dst_r.at[pl.ds(row0, 8)], dstv)
        for j in range(8):
          pltpu.async_copy(tab_ref.at[srcv.at[j]], rows, sem).wait()
          pltpu.sync_copy(rows, spm.at[dstv.at[j]], add=True)
        return carry
      lax.fori_loop(0, n_iter, _it, 0)

      plsc.subcore_barrier()
      pltpu.sync_copy(spm.at[pl.ds(s * wpr, wpr)],
                      out_ref.at[pl.ds(s * wpr, wpr)])

    @pl.when(c == 0)
    def _():
      run(tab_l_r, out_l_r)

    @pl.when(c == 1)
    def _():
      run(tab_r_r, out_r_r)

  fn = pl.kernel(
      body,
      out_type=(out_sds, out_sds),
      mesh=plsc.VectorSubcoreMesh(core_axis_name="c", subcore_axis_name="s"),
      compiler_params=pltpu.CompilerParams(use_tc_tiling_on_sc=False),
      scratch_types=[
          pltpu.VMEM((8, 128), jnp.int32),
          pltpu.VMEM((8, 128), jnp.int32),
          rows_t,
          zbuf_t,
          spm_t,
          pltpu.SemaphoreType.DMA,
      ],
  )
  return fn(tab_l, tab_r, src2d, dst2d)


def _tile(r):
  return 1000 if r % 1000 == 0 else r


def _mm_stats(x, w, x2=None, bias=None, combine_dec=False):
  r, ci = x.shape
  if combine_dec:
    w_sub, w_dec, w_proxy = w
    co = w_dec.shape[1] + 1
    ops = (w_sub, w_dec, w_proxy)
  else:
    co = w.shape[1]
    ops = (w,)
  if bias is None:
    bias = jnp.zeros((1, co), _F32)
  tr = _tile(r)
  grid = r // tr

  def body(*refs):
    if x2 is None:
      x_ref, rest = refs[0], refs[1:]
      xv = x_ref[...]
    else:
      x_ref, x2_ref, rest = refs[0], refs[1], refs[2:]
      xv = x_ref[...] + x2_ref[...]
    if combine_dec:
      ws_ref, wd_ref, wp_ref, b_ref, y_ref, st_ref = rest
      ws = ws_ref[...]
      wd = wd_ref[...]
      ci_in = ws.shape[0]
      wc = jnp.concatenate(
          [jnp.dot(ws, wd[:ws.shape[1], :], preferred_element_type=_F32,
                   precision=jax.lax.Precision.HIGHEST)
           + wd[ws.shape[1]:, :],
           jnp.dot(ws, wp_ref[...], preferred_element_type=_F32,
                   precision=jax.lax.Precision.HIGHEST)], axis=1)
      del ci_in
    else:
      w_ref, b_ref, y_ref, st_ref = rest
      wc = w_ref[...]
    y = jnp.dot(xv, wc, preferred_element_type=_F32,
                   precision=jax.lax.Precision.HIGHEST) + b_ref[...]
    y_ref[...] = y

    @pl.when(pl.program_id(0) == 0)
    def _():
      st_ref[...] = jnp.zeros_like(st_ref)

    st_ref[...] += jnp.concatenate(
        [jnp.sum(y, axis=0)[None, :], jnp.sum(y * y, axis=0)[None, :]], axis=0)

  full = lambda a: pl.BlockSpec(a.shape, lambda i: (0, 0))
  in_specs = [pl.BlockSpec((tr, ci), lambda i: (i, 0))]
  args = [x]
  if x2 is not None:
    in_specs.append(pl.BlockSpec((tr, ci), lambda i: (i, 0)))
    args.append(x2)
  for op_arr in ops:
    in_specs.append(full(op_arr))
    args.append(op_arr)
  in_specs.append(full(bias))
  args.append(bias)

  y, st = pl.pallas_call(
      body,
      grid=(grid,),
      in_specs=in_specs,
      out_specs=[pl.BlockSpec((tr, co), lambda i: (i, 0)),
                 pl.BlockSpec((2, co), lambda i: (0, 0))],
      out_shape=[jax.ShapeDtypeStruct((r, co), _F32),
                 jax.ShapeDtypeStruct((2, co), _F32)],
  )(*args)
  return y, st


def _apply(mode, r, y1=None, st1=None, y2=None, st2=None,
           a=None, sta=None, w1=None):
  rf = float(r)
  tr = _tile(r)

  def nf(y, st):
    mu = st[0:1, :] / rf
    va = st[1:2, :] / rf - mu * mu
    return (y - mu) / jnp.sqrt(va + 1e-5)

  def rank1(av, stav, wv):
    mu = stav[0, 0] / rf
    va = stav[1, 0] / rf - mu * mu
    return (av - mu) * (wv / jnp.sqrt(va * wv * wv + 1e-5))

  def body(*refs):
    i = 0
    if mode in ("n1", "nn", "np", "nr"):
      y1v = refs[i][...]; i += 1
      st1v = refs[i][...]; i += 1
      acc = nf(y1v, st1v)
    if mode == "nn":
      acc = acc + nf(refs[i][...], refs[i + 1][...]); i += 2
    elif mode == "np":
      acc = acc + refs[i][...]; i += 1
    elif mode == "nr" or mode == "r1":
      av = refs[i][...]; stav = refs[i + 1][...]; wv = refs[i + 2][...]
      i += 3
      r1 = rank1(av, stav, wv)
      acc = r1 if mode == "r1" else acc + r1
    refs[i][...] = jnp.maximum(acc, 0.0)

  args, in_specs = [], []

  def add_mat(m):
    args.append(m)
    in_specs.append(pl.BlockSpec((tr, m.shape[1]), lambda i: (i, 0)))

  def add_full(m):
    args.append(m)
    in_specs.append(pl.BlockSpec(m.shape, lambda i: (0, 0)))

  if mode in ("n1", "nn", "np", "nr"):
    add_mat(y1)
    add_full(st1)
    co = y1.shape[1]
  if mode == "nn":
    add_mat(y2)
    add_full(st2)
  elif mode == "np":
    add_mat(y2)
  if mode in ("nr", "r1"):
    add_mat(a)
    add_full(sta)
    add_full(w1)
    if mode == "r1":
      co = w1.shape[1]

  return pl.pallas_call(
      body,
      grid=(r // tr,),
      in_specs=in_specs,
      out_specs=pl.BlockSpec((tr, co), lambda i: (i, 0)),
      out_shape=jax.ShapeDtypeStruct((r, co), _F32),
  )(*args)


def _pad_edges(idx, e_pad, fill):
  e = idx.shape[0]
  if e_pad != e:
    idx = jnp.concatenate(
        [idx, jnp.full((e_pad - e,), fill, jnp.int32)])
  return idx.reshape(-1, 128)


def _halves(m):
  z = jnp.zeros((1, 16), _F32)
  return (jnp.concatenate([m[:, :16], z], 0),
          jnp.concatenate([m[:, 16:], z], 0))


def kernel(x, edge_index_fine, edge_src_down, edge_dst_down,
           edge_index_coarse, batch_size,
           W_d1, W_d2, W_dd, W_e1, W_e2, W_ed, W_e3, W_e4,
           W_sub, W_proxy, b_proxy, W_dec):
  n = x.shape[0]
  m = 12500
  ef = edge_index_fine.shape[1]
  ed = edge_src_down.shape[0]
  ec = edge_index_coarse.shape[1]

  sf, df = edge_index_fine[0], edge_index_fine[1]
  sc_, dc_ = edge_index_coarse[0], edge_index_coarse[1]
  x_in = x[:, 0:1]

  ef_p = _ceil_to(ef, 32 * 1024)
  ed_p = _ceil_to(ed, 16 * 1024)
  ec_p = _ceil_to(ec, 16 * 1024)
  sf2d = _pad_edges(sf, ef_p, n)
  df2d = _pad_edges(df, ef_p, n)
  sd2d = _pad_edges(edge_src_down, ed_p, n)
  dd2d = _pad_edges(edge_dst_down, ed_p, m)
  scc2d = _pad_edges(sc_, ec_p, m)
  dcc2d = _pad_edges(dc_, ec_p, m)
  gu2d = _pad_edges(edge_dst_down, ed_p, m)
  su2d = _pad_edges(edge_src_down, ed_p, n)

  ones11 = jnp.ones((1, 1), _F32)

  x1d = jnp.concatenate([x_in[:, 0], jnp.zeros((1,), _F32)])
  a0p, a1p = _sc_pass(x1d, x1d, sf2d, df2d, width=1, n_out=n,
                      split_edges=True)
  a_sum, st_a = _mm_stats(a0p[:n, None], ones11, x2=a1p[:n, None])
  h = _apply("r1", n, a=a_sum, sta=st_a, w1=W_d1)

  _, st_x = _mm_stats(x_in, ones11)

  h_l, h_r = _halves(h)
  b_l, b_r = _sc_pass(h_l, h_r, sf2d, df2d, width=16, n_out=n)
  b_cat = jnp.concatenate([b_l[:n], b_r[:n]], 1)
  y, st = _mm_stats(b_cat, W_d2)
  enc_in = _apply("nr", n, y, st, a=x_in, sta=st_x, w1=W_dd)

  e_l, e_r = _halves(enc_in)
  c_l, c_r = _sc_pass(e_l, e_r, sd2d, dd2d, width=16, n_out=m)
  c_cat = jnp.concatenate([c_l[:m], c_r[:m]], 1)
  y1, s1 = _mm_stats(c_cat, W_e1)
  y_res, s_res = _mm_stats(c_cat, W_ed)
  h1 = _apply("n1", m, y1, s1)

  h1_l, h1_r = _halves(h1)
  s1_l, s1_r = _sc_pass(h1_l, h1_r, scc2d, dcc2d, width=16, n_out=m)
  s1_cat = jnp.concatenate([s1_l[:m], s1_r[:m]], 1)
  y, st = _mm_stats(s1_cat, W_e2)
  b1 = _apply("nn", m, y, st, y_res, s_res)

  b1_l, b1_r = _halves(b1)
  s2_l, s2_r = _sc_pass(b1_l, b1_r, scc2d, dcc2d, width=16, n_out=m)
  s2_cat = jnp.concatenate([s2_l[:m], s2_r[:m]], 1)
  y, st = _mm_stats(s2_cat, W_e3)
  h2 = _apply("n1", m, y, st)

  h2_l, h2_r = _halves(h2)
  s3_l, s3_r = _sc_pass(h2_l, h2_r, scc2d, dcc2d, width=16, n_out=m)
  s3_cat = jnp.concatenate([s3_l[:m], s3_r[:m]], 1)
  y, st = _mm_stats(s3_cat, W_e4)
  encoded = _apply("np", m, y, st, y2=b1)

  bias_cat = jnp.concatenate([jnp.zeros((1, 32), _F32),
                              b_proxy.reshape(1, 1)], axis=1)
  yc, _ = _mm_stats(encoded, (W_sub, W_dec, W_proxy), bias=bias_cat,
                    combine_dec=True)
  z = yc[:, :32]
  proxy = yc[:, 32:33]

  z_l, z_r = _halves(z)
  u_l, u_r = _sc_pass(z_l, z_r, gu2d, su2d, width=16, n_out=n)
  u_cat = jnp.concatenate([u_l[:n], u_r[:n]], 1)
  yu, stu = _mm_stats(u_cat, jnp.eye(32, dtype=_F32))
  output = _apply("n1", n, yu, stu)
  return (output, proxy)


# speedup vs baseline: 9.0245x; 1.1575x over previous
import functools

import jax
import jax.numpy as jnp
from jax import lax
from jax.experimental import pallas as pl
from jax.experimental.pallas import tpu as pltpu
from jax.experimental.pallas import tpu_sc as plsc

_F32 = jnp.float32
_NJ = 4


def _ceil_to(a, b):
  return -(-a // b) * b


def _sc_pass(tab_l, tab_r, src2d, dst2d, *, width, n_out, split_edges=False):
  e_total = src2d.shape[0] * 128
  n_workers = 32 if split_edges else 16
  epw = e_total // n_workers
  n_iter = epw // (_NJ * 128)
  zr = 256
  rows_per_sub = -(-(n_out + 1) // 16)
  nz = -(-rows_per_sub // zr)
  wpr = nz * zr
  op = 16 * wpr

  assert n_iter % 2 == 0, "edge padding must make the pipelined loop even"

  if width == 1:
    rows_t = pltpu.VMEM((2 * _NJ, 128), _F32)
    zbuf_t = pltpu.VMEM((zr,), _F32)
    spm_t = pltpu.VMEM_SHARED((op,), _F32)
    out_sds = jax.ShapeDtypeStruct((op,), _F32)
  else:
    rows_t = pltpu.VMEM((2 * _NJ, 128, width), _F32)
    zbuf_t = pltpu.VMEM((zr, width), _F32)
    spm_t = pltpu.VMEM_SHARED((op, width), _F32)
    out_sds = jax.ShapeDtypeStruct((op, width), _F32)

  def body(tab_l_r, tab_r_r, src_r, dst_r, out_l_r, out_r_r,
           srcv, dstv, rows, zbuf, spm, sem_g0, sem_g1, sem_s0, sem_s1):
    sem_g = (sem_g0, sem_g1)
    sem_s = (sem_s0, sem_s1)
    c = lax.axis_index("c")
    s = lax.axis_index("s")

    if width == 1:
      def _zb(i, carry):
        zbuf[pl.ds(16 * i, 16)] = jnp.zeros((16,), _F32)
        return carry
      lax.fori_loop(0, zr // 16, _zb, 0)
    else:
      def _zb(i, carry):
        zbuf[i, :] = jnp.zeros((16,), _F32)
        return carry
      lax.fori_loop(0, zr, _zb, 0)

    def run(tab_ref, out_ref):
      def _zc(z, carry):
        pltpu.sync_copy(zbuf, spm.at[pl.ds((s * nz + z) * zr, zr)])
        return carry
      lax.fori_loop(0, nz, _zc, 0)
      plsc.subcore_barrier()

      wid = c * 16 + s if split_edges else s
      base_row = wid * (epw // 128)

      def load_idx(i, bank):
        pltpu.sync_copy(src_r.at[pl.ds(base_row + i * _NJ, _NJ)],
                        srcv.at[pl.ds(bank * _NJ, _NJ)])
        pltpu.sync_copy(dst_r.at[pl.ds(base_row + i * _NJ, _NJ)],
                        dstv.at[pl.ds(bank * _NJ, _NJ)])

      def fire_gathers(bank):
        for j in range(_NJ):
          k = bank * _NJ + j
          pltpu.async_copy(tab_ref.at[srcv.at[k]], rows.at[k], sem_g[bank])

      def wait_gathers(bank):
        for j in range(_NJ):
          k = bank * _NJ + j
          pltpu.make_async_copy(tab_ref.at[srcv.at[k]], rows.at[k],
                                sem_g[bank]).wait()

      def fire_scatters(bank):
        for j in range(_NJ):
          k = bank * _NJ + j
          pltpu.async_copy(rows.at[k], spm.at[dstv.at[k]], sem_s[bank],
                           add=True)

      def wait_scatters(bank):
        for j in range(_NJ):
          k = bank * _NJ + j
          pltpu.make_async_copy(rows.at[k], spm.at[dstv.at[k]],
                                sem_s[bank]).wait()

      load_idx(0, 0)
      fire_gathers(0)

      def _it(i2, carry):
        for p in (0, 1):
          b = p
          nb = 1 - p
          i = 2 * i2 + p

          @pl.when(i >= 1)
          def _():
            wait_scatters(nb)

          @pl.when(i + 1 < n_iter)
          def _():
            load_idx(i + 1, nb)
            fire_gathers(nb)

          wait_gathers(b)
          fire_scatters(b)
        return carry
      lax.fori_loop(0, n_iter // 2, _it, 0)
      wait_scatters(1)

      plsc.subcore_barrier()
      pltpu.sync_copy(spm.at[pl.ds(s * wpr, wpr)],
                      out_ref.at[pl.ds(s * wpr, wpr)])

    @pl.when(c == 0)
    def _():
      run(tab_l_r, out_l_r)

    @pl.when(c == 1)
    def _():
      run(tab_r_r, out_r_r)

  fn = pl.kernel(
      body,
      out_type=(out_sds, out_sds),
      mesh=plsc.VectorSubcoreMesh(core_axis_name="c", subcore_axis_name="s"),
      compiler_params=pltpu.CompilerParams(use_tc_tiling_on_sc=False),
      scratch_types=[
          pltpu.VMEM((2 * _NJ, 128), jnp.int32),
          pltpu.VMEM((2 * _NJ, 128), jnp.int32),
          rows_t,
          zbuf_t,
          spm_t,
          pltpu.SemaphoreType.DMA,
          pltpu.SemaphoreType.DMA,
          pltpu.SemaphoreType.DMA,
          pltpu.SemaphoreType.DMA,
      ],
  )
  return fn(tab_l, tab_r, src2d, dst2d)


def _tile(r):
  return 1000 if r % 1000 == 0 else r


def _mm_stats(x, w, x2=None, bias=None, combine_dec=False):
  r, ci = x.shape
  if combine_dec:
    w_sub, w_dec, w_proxy = w
    co = w_dec.shape[1] + 1
    ops = (w_sub, w_dec, w_proxy)
  else:
    co = w.shape[1]
    ops = (w,)
  if bias is None:
    bias = jnp.zeros((1, co), _F32)
  tr = _tile(r)
  grid = r // tr

  def body(*refs):
    if x2 is None:
      x_ref, rest = refs[0], refs[1:]
      xv = x_ref[...]
    else:
      x_ref, x2_ref, rest = refs[0], refs[1], refs[2:]
      xv = x_ref[...] + x2_ref[...]
    if combine_dec:
      ws_ref, wd_ref, wp_ref, b_ref, y_ref, st_ref = rest
      ws = ws_ref[...]
      wd = wd_ref[...]
      ci_in = ws.shape[0]
      wc = jnp.concatenate(
          [jnp.dot(ws, wd[:ws.shape[1], :], preferred_element_type=_F32,
                   precision=jax.lax.Precision.HIGHEST)
           + wd[ws.shape[1]:, :],
           jnp.dot(ws, wp_ref[...], preferred_element_type=_F32,
                   precision=jax.lax.Precision.HIGHEST)], axis=1)
      del ci_in
    else:
      w_ref, b_ref, y_ref, st_ref = rest
      wc = w_ref[...]
    y = jnp.dot(xv, wc, preferred_element_type=_F32,
                   precision=jax.lax.Precision.HIGHEST) + b_ref[...]
    y_ref[...] = y

    @pl.when(pl.program_id(0) == 0)
    def _():
      st_ref[...] = jnp.zeros_like(st_ref)

    st_ref[...] += jnp.concatenate(
        [jnp.sum(y, axis=0)[None, :], jnp.sum(y * y, axis=0)[None, :]], axis=0)

  full = lambda a: pl.BlockSpec(a.shape, lambda i: (0, 0))
  in_specs = [pl.BlockSpec((tr, ci), lambda i: (i, 0))]
  args = [x]
  if x2 is not None:
    in_specs.append(pl.BlockSpec((tr, ci), lambda i: (i, 0)))
    args.append(x2)
  for op_arr in ops:
    in_specs.append(full(op_arr))
    args.append(op_arr)
  in_specs.append(full(bias))
  args.append(bias)

  y, st = pl.pallas_call(
      body,
      grid=(grid,),
      in_specs=in_specs,
      out_specs=[pl.BlockSpec((tr, co), lambda i: (i, 0)),
                 pl.BlockSpec((2, co), lambda i: (0, 0))],
      out_shape=[jax.ShapeDtypeStruct((r, co), _F32),
                 jax.ShapeDtypeStruct((2, co), _F32)],
  )(*args)
  return y, st


def _apply(mode, r, y1=None, st1=None, y2=None, st2=None,
           a=None, sta=None, w1=None):
  rf = float(r)
  tr = _tile(r)

  def nf(y, st):
    mu = st[0:1, :] / rf
    va = st[1:2, :] / rf - mu * mu
    return (y - mu) / jnp.sqrt(va + 1e-5)

  def rank1(av, stav, wv):
    mu = stav[0, 0] / rf
    va = stav[1, 0] / rf - mu * mu
    return (av - mu) * (wv / jnp.sqrt(va * wv * wv + 1e-5))

  def body(*refs):
    i = 0
    if mode in ("n1", "nn", "np", "nr"):
      y1v = refs[i][...]; i += 1
      st1v = refs[i][...]; i += 1
      acc = nf(y1v, st1v)
    if mode == "nn":
      acc = acc + nf(refs[i][...], refs[i + 1][...]); i += 2
    elif mode == "np":
      acc = acc + refs[i][...]; i += 1
    elif mode == "nr" or mode == "r1":
      av = refs[i][...]; stav = refs[i + 1][...]; wv = refs[i + 2][...]
      i += 3
      r1 = rank1(av, stav, wv)
      acc = r1 if mode == "r1" else acc + r1
    refs[i][...] = jnp.maximum(acc, 0.0)

  args, in_specs = [], []

  def add_mat(m):
    args.append(m)
    in_specs.append(pl.BlockSpec((tr, m.shape[1]), lambda i: (i, 0)))

  def add_full(m):
    args.append(m)
    in_specs.append(pl.BlockSpec(m.shape, lambda i: (0, 0)))

  if mode in ("n1", "nn", "np", "nr"):
    add_mat(y1)
    add_full(st1)
    co = y1.shape[1]
  if mode == "nn":
    add_mat(y2)
    add_full(st2)
  elif mode == "np":
    add_mat(y2)
  if mode in ("nr", "r1"):
    add_mat(a)
    add_full(sta)
    add_full(w1)
    if mode == "r1":
      co = w1.shape[1]

  return pl.pallas_call(
      body,
      grid=(r // tr,),
      in_specs=in_specs,
      out_specs=pl.BlockSpec((tr, co), lambda i: (i, 0)),
      out_shape=jax.ShapeDtypeStruct((r, co), _F32),
  )(*args)


def _pad_edges(idx, e_pad, fill):
  e = idx.shape[0]
  if e_pad != e:
    idx = jnp.concatenate(
        [idx, jnp.full((e_pad - e,), fill, jnp.int32)])
  return idx.reshape(-1, 128)


def _halves(m):
  z = jnp.zeros((1, 16), _F32)
  return (jnp.concatenate([m[:, :16], z], 0),
          jnp.concatenate([m[:, 16:], z], 0))


def kernel(x, edge_index_fine, edge_src_down, edge_dst_down,
           edge_index_coarse, batch_size,
           W_d1, W_d2, W_dd, W_e1, W_e2, W_ed, W_e3, W_e4,
           W_sub, W_proxy, b_proxy, W_dec):
  n = x.shape[0]
  m = 12500
  ef = edge_index_fine.shape[1]
  ed = edge_src_down.shape[0]
  ec = edge_index_coarse.shape[1]

  sf, df = edge_index_fine[0], edge_index_fine[1]
  sc_, dc_ = edge_index_coarse[0], edge_index_coarse[1]
  x_in = x[:, 0:1]

  ef_p = _ceil_to(ef, 64 * 1024)
  ed_p = _ceil_to(ed, 32 * 1024)
  ec_p = _ceil_to(ec, 32 * 1024)
  sf2d = _pad_edges(sf, ef_p, n)
  df2d = _pad_edges(df, ef_p, n)
  sd2d = _pad_edges(edge_src_down, ed_p, n)
  dd2d = _pad_edges(edge_dst_down, ed_p, m)
  scc2d = _pad_edges(sc_, ec_p, m)
  dcc2d = _pad_edges(dc_, ec_p, m)
  gu2d = _pad_edges(edge_dst_down, ed_p, m)
  su2d = _pad_edges(edge_src_down, ed_p, n)

  ones11 = jnp.ones((1, 1), _F32)

  x1d = jnp.concatenate([x_in[:, 0], jnp.zeros((1,), _F32)])
  a0p, a1p = _sc_pass(x1d, x1d, sf2d, df2d, width=1, n_out=n,
                      split_edges=True)
  a_sum, st_a = _mm_stats(a0p[:n, None], ones11, x2=a1p[:n, None])
  h = _apply("r1", n, a=a_sum, sta=st_a, w1=W_d1)

  _, st_x = _mm_stats(x_in, ones11)

  h_l, h_r = _halves(h)
  b_l, b_r = _sc_pass(h_l, h_r, sf2d, df2d, width=16, n_out=n)
  b_cat = jnp.concatenate([b_l[:n], b_r[:n]], 1)
  y, st = _mm_stats(b_cat, W_d2)
  enc_in = _apply("nr", n, y, st, a=x_in, sta=st_x, w1=W_dd)

  e_l, e_r = _halves(enc_in)
  c_l, c_r = _sc_pass(e_l, e_r, sd2d, dd2d, width=16, n_out=m)
  c_cat = jnp.concatenate([c_l[:m], c_r[:m]], 1)
  y1, s1 = _mm_stats(c_cat, W_e1)
  y_res, s_res = _mm_stats(c_cat, W_ed)
  h1 = _apply("n1", m, y1, s1)

  h1_l, h1_r = _halves(h1)
  s1_l, s1_r = _sc_pass(h1_l, h1_r, scc2d, dcc2d, width=16, n_out=m)
  s1_cat = jnp.concatenate([s1_l[:m], s1_r[:m]], 1)
  y, st = _mm_stats(s1_cat, W_e2)
  b1 = _apply("nn", m, y, st, y_res, s_res)

  b1_l, b1_r = _halves(b1)
  s2_l, s2_r = _sc_pass(b1_l, b1_r, scc2d, dcc2d, width=16, n_out=m)
  s2_cat = jnp.concatenate([s2_l[:m], s2_r[:m]], 1)
  y, st = _mm_stats(s2_cat, W_e3)
  h2 = _apply("n1", m, y, st)

  h2_l, h2_r = _halves(h2)
  s3_l, s3_r = _sc_pass(h2_l, h2_r, scc2d, dcc2d, width=16, n_out=m)
  s3_cat = jnp.concatenate([s3_l[:m], s3_r[:m]], 1)
  y, st = _mm_stats(s3_cat, W_e4)
  encoded = _apply("np", m, y, st, y2=b1)

  bias_cat = jnp.concatenate([jnp.zeros((1, 32), _F32),
                              b_proxy.reshape(1, 1)], axis=1)
  yc, _ = _mm_stats(encoded, (W_sub, W_dec, W_proxy), bias=bias_cat,
                    combine_dec=True)
  z = yc[:, :32]
  proxy = yc[:, 32:33]

  z_l, z_r = _halves(z)
  u_l, u_r = _sc_pass(z_l, z_r, gu2d, su2d, width=16, n_out=n)
  u_cat = jnp.concatenate([u_l[:n], u_r[:n]], 1)
  yu, stu = _mm_stats(u_cat, jnp.eye(32, dtype=_F32))
  output = _apply("n1", n, yu, stu)
  return (output, proxy)
